# baseline (speedup 1.0000x reference)
import jax
import jax.numpy as jnp
import numpy as np
from jax import lax
from jax.experimental import pallas as pl
from jax.experimental.pallas import tpu as pltpu
from jax.experimental.pallas import tpu_sc as plsc

_ATTR_DIM = 64
_NFREQ = 3
_MU = 11.0
_B = 4096
_S = 200
_NWORKERS = 32
_BATCH_PER_W = _B // _NWORKERS
_EPC = 2
_CR = _EPC * _S
_NCHUNKS = _BATCH_PER_W // _EPC
_FWIDTH = 16
_SLABS = ((0, 128), (128, 72))


def _fourier_table() -> np.ndarray:
    b = np.arange(256)
    xi = ((b >> 4) & 15).astype(np.float32)
    yi = (b & 15).astype(np.float32)
    xn = xi / np.float32(_MU - 1.0) * np.float32(2.0) - np.float32(1.0)
    yn = yi / np.float32(_MU - 1.0) * np.float32(2.0) - np.float32(1.0)
    freqs = (2.0 ** np.arange(_NFREQ)).astype(np.float32)
    xs = xn[:, None] * freqs[None, :]
    ys = yn[:, None] * freqs[None, :]
    t = np.zeros((256, _FWIDTH), dtype=np.float32)
    t[:, 0:3] = np.cos(xs)
    t[:, 3:6] = np.sin(xs)
    t[:, 6:9] = np.cos(ys)
    t[:, 9:12] = np.sin(ys)
    return t


_TXY = _fourier_table()


def _sc_body(td_hbm, w_hbm, txy_hbm, out_hbm,
             td_v, idx_a, embed_v, four_p, txy_v, sem_a):
    wid = lax.axis_index("s") * 2 + lax.axis_index("c")
    lanes = lax.iota(jnp.int32, 16)
    c0 = jnp.zeros((16,), jnp.int32)
    c1 = jnp.full((16,), 1, jnp.int32)
    c2 = jnp.full((16,), 2, jnp.int32)
    c12 = jnp.full((16,), 12, jnp.int32)
    pltpu.sync_copy(txy_hbm, txy_v)

    def chunk(t, _):
        base = wid * _BATCH_PER_W + t * _EPC
        pltpu.sync_copy(td_hbm.at[pl.ds(base, _EPC)], td_v)

        def grp_idx(g, _):
            r = g * 16 + lanes
            e = (r >= _S).astype(jnp.int32)
            rr = r - e * _S
            a = plsc.load_gather(td_v, [e, rr, c1])
            idx_a[pl.ds(g * 16, 16)] = a & 255
            return 0

        lax.fori_loop(0, _CR // 16, grp_idx, 0)
        copies = []
        for e in range(_EPC):
            for off, sz in _SLABS:
                copies.append(pltpu.async_copy(
                    w_hbm.at[idx_a.at[pl.ds(e * _S + off, sz)]],
                    embed_v.at[e, pl.ds(off, sz)], sem_a))

        def grp_four(g, _):
            r = g * 16 + lanes
            e = (r >= _S).astype(jnp.int32)
            rr = r - e * _S
            b = plsc.load_gather(td_v, [e, rr, c0]) & 255
            for c in range(12):
                col = jnp.full((16,), c, jnp.int32)
                fc = plsc.load_gather(txy_v, [b, col])
                plsc.store_scatter(four_p, [e, rr, col], fc)
            v = plsc.load_gather(td_v, [e, rr, c2]).astype(jnp.float32)
            plsc.store_scatter(four_p, [e, rr, c12], v)
            return 0

        lax.fori_loop(0, _CR // 16, grp_four, 0)
        for cp in copies:
            cp.wait()
        pltpu.sync_copy(
            embed_v,
            out_hbm.at[pl.ds(base, _EPC), pl.ds(0, _S), pl.ds(0, 64)])
        pltpu.sync_copy(
            four_p,
            out_hbm.at[pl.ds(base, _EPC), pl.ds(0, _S), pl.ds(64, 13)])
        return 0

    lax.fori_loop(0, _NCHUNKS, chunk, 0)


@jax.jit
def _run(td, w, txy):
    mesh = plsc.VectorSubcoreMesh(core_axis_name="c", subcore_axis_name="s")
    f = pl.kernel(
        _sc_body,
        out_type=jax.ShapeDtypeStruct((_B, _S, 77), jnp.float32),
        mesh=mesh,
        scratch_types=[
            pltpu.VMEM((_EPC, _S, 3), jnp.int32),
            pltpu.VMEM((_CR,), jnp.int32),
            pltpu.VMEM((_EPC, _S, _ATTR_DIM), jnp.float32),
            pltpu.VMEM((_EPC, _S, 13), jnp.float32),
            pltpu.VMEM((256, _FWIDTH), jnp.float32),
            pltpu.SemaphoreType.DMA,
        ],
        compiler_params=pltpu.CompilerParams(
            use_tc_tiling_on_sc=False, needs_layout_passes=False),
    )
    return f(td, w, txy)


def kernel(td, W):
    return _run(td, W, jnp.asarray(_TXY))


# speedup vs baseline: 2.7312x; 2.7312x over previous
import jax
import jax.numpy as jnp
import numpy as np
from jax import lax
from jax.experimental import pallas as pl
from jax.experimental.pallas import tpu as pltpu
from jax.experimental.pallas import tpu_sc as plsc

_NFREQ = 3
_MU = 11.0
_B = 4096
_S = 200
_N = _B * _S
_NWORKERS = 32
_EPW = _N // _NWORKERS
_E = 512
_NCHUNKS = _EPW // _E
_FWIDTH = 16


def _fourier_table() -> np.ndarray:
    b = np.arange(256)
    xi = ((b >> 4) & 15).astype(np.float32)
    yi = (b & 15).astype(np.float32)
    xn = xi / np.float32(_MU - 1.0) * np.float32(2.0) - np.float32(1.0)
    yn = yi / np.float32(_MU - 1.0) * np.float32(2.0) - np.float32(1.0)
    freqs = (2.0 ** np.arange(_NFREQ)).astype(np.float32)
    xs = xn[:, None] * freqs[None, :]
    ys = yn[:, None] * freqs[None, :]
    t = np.zeros((256, _FWIDTH), dtype=np.float32)
    t[:, 0:3] = np.cos(xs)
    t[:, 3:6] = np.sin(xs)
    t[:, 6:9] = np.cos(ys)
    t[:, 9:12] = np.sin(ys)
    return t


_TXY = _fourier_table()


def _sc_body(td_hbm, w_hbm, txy_hbm, out_hbm, td_v, out_v, w_v, txy_v, sem):
    wid = lax.axis_index("s") * 2 + lax.axis_index("c")
    pltpu.sync_copy(w_hbm, w_v)
    pltpu.sync_copy(txy_hbm, txy_v)

    def chunk(t, _):
        n0 = wid * _EPW + t * _E
        pltpu.sync_copy(td_hbm.at[:, pl.ds(n0, _E)], td_v)

        def grp(g, _):
            o = g * 16
            attr = td_v[1, pl.ds(o, 16)] & 255
            byte = td_v[0, pl.ds(o, 16)] & 255
            for p in range(64):
                pc = jnp.full((16,), p, jnp.int32)
                out_v[p, pl.ds(o, 16)] = plsc.load_gather(w_v, [attr, pc])
            for p in range(12):
                pc = jnp.full((16,), p, jnp.int32)
                out_v[64 + p, pl.ds(o, 16)] = plsc.load_gather(
                    txy_v, [byte, pc])
            out_v[76, pl.ds(o, 16)] = td_v[2, pl.ds(o, 16)].astype(jnp.float32)
            return 0

        lax.fori_loop(0, _E // 16, grp, 0)
        pltpu.sync_copy(out_v, out_hbm.at[:, pl.ds(n0, _E)])
        return 0

    lax.fori_loop(0, _NCHUNKS, chunk, 0)


@jax.jit
def _run(td_lin, w, txy):
    mesh = plsc.VectorSubcoreMesh(core_axis_name="c", subcore_axis_name="s")
    f = pl.kernel(
        _sc_body,
        out_type=jax.ShapeDtypeStruct((77, _N), jnp.float32),
        mesh=mesh,
        scratch_types=[
            pltpu.VMEM((3, _E), jnp.int32),
            pltpu.VMEM((77, _E), jnp.float32),
            pltpu.VMEM((256, 64), jnp.float32),
            pltpu.VMEM((256, _FWIDTH), jnp.float32),
            pltpu.SemaphoreType.DMA,
        ],
        compiler_params=pltpu.CompilerParams(
            use_tc_tiling_on_sc=False, needs_layout_passes=False),
    )
    return f(td_lin, w, txy)


def kernel(td, W):
    td_lin = (td.transpose(2, 1, 0)
                .reshape(3, _S // 8, 8, _B // 128, 128)
                .transpose(0, 1, 3, 2, 4)
                .reshape(3, _N))
    k5 = _run(td_lin, W, jnp.asarray(_TXY))
    out = (k5.reshape(77, _S // 8, _B // 128, 8, 128)
             .transpose(2, 4, 1, 3, 0)
             .reshape(_B, _S, 77))
    return out


# speedup vs baseline: 3.9354x; 1.4409x over previous
import jax
import jax.numpy as jnp
import numpy as np
from jax import lax
from jax.experimental import pallas as pl
from jax.experimental.pallas import tpu as pltpu
from jax.experimental.pallas import tpu_sc as plsc

_NFREQ = 3
_MU = 11.0
_B = 4096
_S = 200
_N = _B * _S
_NWORKERS = 32
_EPW = _N // _NWORKERS
_E = 512
_NCHUNKS = _EPW // _E
_FWIDTH = 16


def _fourier_table() -> np.ndarray:
    b = np.arange(256)
    xi = ((b >> 4) & 15).astype(np.float32)
    yi = (b & 15).astype(np.float32)
    xn = xi / np.float32(_MU - 1.0) * np.float32(2.0) - np.float32(1.0)
    yn = yi / np.float32(_MU - 1.0) * np.float32(2.0) - np.float32(1.0)
    freqs = (2.0 ** np.arange(_NFREQ)).astype(np.float32)
    xs = xn[:, None] * freqs[None, :]
    ys = yn[:, None] * freqs[None, :]
    t = np.zeros((256, _FWIDTH), dtype=np.float32)
    t[:, 0:3] = np.cos(xs)
    t[:, 3:6] = np.sin(xs)
    t[:, 6:9] = np.cos(ys)
    t[:, 9:12] = np.sin(ys)
    return t


_TXY = _fourier_table()


def _sc_body(td_hbm, w_hbm, txy_hbm, out_hbm, td_v, out_v, w_v, txy_v, sem):
    wid = lax.axis_index("s") * 2 + lax.axis_index("c")
    pltpu.sync_copy(w_hbm, w_v)
    pltpu.sync_copy(txy_hbm, txy_v)

    def chunk(t, _):
        n0 = wid * _EPW + t * _E
        pltpu.sync_copy(td_hbm.at[:, pl.ds(n0, _E)], td_v)

        @plsc.parallel_loop(0, _E, step=16, unroll=2)
        def grp(o):
            attr = td_v[1, pl.ds(o, 16)] & 255
            byte = td_v[0, pl.ds(o, 16)] & 255
            for p in range(64):
                pc = jnp.full((16,), p, jnp.int32)
                out_v[p, pl.ds(o, 16)] = plsc.load_gather(w_v, [attr, pc])
            for p in range(12):
                pc = jnp.full((16,), p, jnp.int32)
                out_v[64 + p, pl.ds(o, 16)] = plsc.load_gather(
                    txy_v, [byte, pc])
            out_v[76, pl.ds(o, 16)] = td_v[2, pl.ds(o, 16)].astype(jnp.float32)
        pltpu.sync_copy(out_v, out_hbm.at[:, pl.ds(n0, _E)])
        return 0

    lax.fori_loop(0, _NCHUNKS, chunk, 0)


@jax.jit
def _run(td_lin, w, txy):
    mesh = plsc.VectorSubcoreMesh(core_axis_name="c", subcore_axis_name="s")
    f = pl.kernel(
        _sc_body,
        out_type=jax.ShapeDtypeStruct((77, _N), jnp.float32),
        mesh=mesh,
        scratch_types=[
            pltpu.VMEM((3, _E), jnp.int32),
            pltpu.VMEM((77, _E), jnp.float32),
            pltpu.VMEM((256, 64), jnp.float32),
            pltpu.VMEM((256, _FWIDTH), jnp.float32),
            pltpu.SemaphoreType.DMA,
        ],
        compiler_params=pltpu.CompilerParams(
            use_tc_tiling_on_sc=False, needs_layout_passes=False),
    )
    return f(td_lin, w, txy)


def kernel(td, W):
    td_lin = (td.transpose(2, 1, 0)
                .reshape(3, _S // 8, 8, _B // 128, 128)
                .transpose(0, 1, 3, 2, 4)
                .reshape(3, _N))
    k5 = _run(td_lin, W, jnp.asarray(_TXY))
    out = (k5.reshape(77, _S // 8, _B // 128, 8, 128)
             .transpose(2, 4, 1, 3, 0)
             .reshape(_B, _S, 77))
    return out


# speedup vs baseline: 4.5564x; 1.1578x over previous
import jax
import jax.numpy as jnp
import numpy as np
from jax import lax
from jax.experimental import pallas as pl
from jax.experimental.pallas import tpu as pltpu
from jax.experimental.pallas import tpu_sc as plsc

_NFREQ = 3
_MU = 11.0
_B = 4096
_S = 200
_N = _B * _S
_NWORKERS = 32
_EPW = _N // _NWORKERS
_E = 512
_NCHUNKS = _EPW // _E
_FWIDTH = 16


def _fourier_table() -> np.ndarray:
    b = np.arange(256)
    xi = ((b >> 4) & 15).astype(np.float32)
    yi = (b & 15).astype(np.float32)
    xn = xi / np.float32(_MU - 1.0) * np.float32(2.0) - np.float32(1.0)
    yn = yi / np.float32(_MU - 1.0) * np.float32(2.0) - np.float32(1.0)
    freqs = (2.0 ** np.arange(_NFREQ)).astype(np.float32)
    xs = xn[:, None] * freqs[None, :]
    ys = yn[:, None] * freqs[None, :]
    t = np.zeros((256, _FWIDTH), dtype=np.float32)
    t[:, 0:3] = np.cos(xs)
    t[:, 3:6] = np.sin(xs)
    t[:, 6:9] = np.cos(ys)
    t[:, 9:12] = np.sin(ys)
    return t


_TXY = _fourier_table()


def _sc_body(td_hbm, w_hbm, txy_hbm, out_hbm, td_v, out_v, w_v, txy_v, sem):
    wid = lax.axis_index("s") * 2 + lax.axis_index("c")
    pltpu.sync_copy(w_hbm, w_v)
    pltpu.sync_copy(txy_hbm, txy_v)

    def chunk(t, _):
        n0 = wid * _EPW + t * _E
        pltpu.sync_copy(td_hbm.at[:, pl.ds(n0, _E)], td_v)

        @plsc.parallel_loop(0, _E, step=16, unroll=4)
        def grp(o):
            attr = td_v[1, pl.ds(o, 16)] & 255
            byte = td_v[0, pl.ds(o, 16)] & 255
            for p in range(64):
                pc = jnp.full((16,), p, jnp.int32)
                out_v[p, pl.ds(o, 16)] = plsc.load_gather(w_v, [attr, pc])
            for p in range(12):
                pc = jnp.full((16,), p, jnp.int32)
                out_v[64 + p, pl.ds(o, 16)] = plsc.load_gather(
                    txy_v, [byte, pc])
            out_v[76, pl.ds(o, 16)] = td_v[2, pl.ds(o, 16)].astype(jnp.float32)
        pltpu.sync_copy(out_v, out_hbm.at[:, pl.ds(n0, _E)])
        return 0

    lax.fori_loop(0, _NCHUNKS, chunk, 0)


@jax.jit
def _run(td_lin, w, txy):
    mesh = plsc.VectorSubcoreMesh(core_axis_name="c", subcore_axis_name="s")
    f = pl.kernel(
        _sc_body,
        out_type=jax.ShapeDtypeStruct((77, _N), jnp.float32),
        mesh=mesh,
        scratch_types=[
            pltpu.VMEM((3, _E), jnp.int32),
            pltpu.VMEM((77, _E), jnp.float32),
            pltpu.VMEM((256, 64), jnp.float32),
            pltpu.VMEM((256, _FWIDTH), jnp.float32),
            pltpu.SemaphoreType.DMA,
        ],
        compiler_params=pltpu.CompilerParams(
            use_tc_tiling_on_sc=False, needs_layout_passes=False),
    )
    return f(td_lin, w, txy)


def kernel(td, W):
    td_lin = (td.transpose(2, 1, 0)
                .reshape(3, _S // 8, 8, _B // 128, 128)
                .transpose(0, 1, 3, 2, 4)
                .reshape(3, _N))
    k5 = _run(td_lin, W, jnp.asarray(_TXY))
    out = (k5.reshape(77, _S // 8, _B // 128, 8, 128)
             .transpose(2, 4, 1, 3, 0)
             .reshape(_B, _S, 77))
    return out


# speedup vs baseline: 4.8009x; 1.0537x over previous
import jax
import jax.numpy as jnp
import numpy as np
from jax import lax
from jax.experimental import pallas as pl
from jax.experimental.pallas import tpu as pltpu
from jax.experimental.pallas import tpu_sc as plsc

_NFREQ = 3
_MU = 11.0
_B = 4096
_S = 200
_N = _B * _S
_NWORKERS = 32
_EPW = _N // _NWORKERS
_E = 512
_NCHUNKS = _EPW // _E
_FWIDTH = 16


def _fourier_table() -> np.ndarray:
    b = np.arange(256)
    xi = ((b >> 4) & 15).astype(np.float32)
    yi = (b & 15).astype(np.float32)
    xn = xi / np.float32(_MU - 1.0) * np.float32(2.0) - np.float32(1.0)
    yn = yi / np.float32(_MU - 1.0) * np.float32(2.0) - np.float32(1.0)
    freqs = (2.0 ** np.arange(_NFREQ)).astype(np.float32)
    xs = xn[:, None] * freqs[None, :]
    ys = yn[:, None] * freqs[None, :]
    t = np.zeros((256, _FWIDTH), dtype=np.float32)
    t[:, 0:3] = np.cos(xs)
    t[:, 3:6] = np.sin(xs)
    t[:, 6:9] = np.cos(ys)
    t[:, 9:12] = np.sin(ys)
    return t


_TXY = _fourier_table()


def _sc_body(td_hbm, w_hbm, txy_hbm, out_hbm, td_v, out_v, w_v, txy_v,
             sin0, sin1, sout0, sout1):
    wid = lax.axis_index("s") * 2 + lax.axis_index("c")
    base = wid * _EPW
    sins = (sin0, sin1)
    souts = (sout0, sout1)
    pltpu.sync_copy(w_hbm, w_v)
    pltpu.sync_copy(txy_hbm, txy_v)
    pltpu.async_copy(td_hbm.at[:, pl.ds(base, _E)], td_v.at[0], sins[0])

    def pair(tp, _):
        for b in range(2):
            t = 2 * tp + b
            n0 = base + t * _E
            pltpu.make_async_copy(
                td_hbm.at[:, pl.ds(n0, _E)], td_v.at[b], sins[b]).wait()

            @pl.when(t + 1 < _NCHUNKS)
            def _():
                pltpu.async_copy(td_hbm.at[:, pl.ds(n0 + _E, _E)],
                                 td_v.at[1 - b], sins[1 - b])

            @pl.when(t >= 2)
            def _():
                pltpu.make_async_copy(
                    out_v.at[b], out_hbm.at[:, pl.ds(n0 - 2 * _E, _E)],
                    souts[b]).wait()

            @plsc.parallel_loop(0, _E, step=16, unroll=4)
            def grp(o):
                attr = td_v[b, 1, pl.ds(o, 16)] & 255
                byte = td_v[b, 0, pl.ds(o, 16)] & 255
                for p in range(64):
                    pc = jnp.full((16,), p, jnp.int32)
                    out_v[b, p, pl.ds(o, 16)] = plsc.load_gather(
                        w_v, [attr, pc])
                for p in range(12):
                    pc = jnp.full((16,), p, jnp.int32)
                    out_v[b, 64 + p, pl.ds(o, 16)] = plsc.load_gather(
                        txy_v, [byte, pc])
                out_v[b, 76, pl.ds(o, 16)] = (
                    td_v[b, 2, pl.ds(o, 16)].astype(jnp.float32))

            pltpu.async_copy(out_v.at[b], out_hbm.at[:, pl.ds(n0, _E)],
                             souts[b])
        return 0

    lax.fori_loop(0, _NCHUNKS // 2, pair, 0)
    end = base + _NCHUNKS * _E
    pltpu.make_async_copy(
        out_v.at[0], out_hbm.at[:, pl.ds(end - 2 * _E, _E)], souts[0]).wait()
    pltpu.make_async_copy(
        out_v.at[1], out_hbm.at[:, pl.ds(end - _E, _E)], souts[1]).wait()


@jax.jit
def _run(td_lin, w, txy):
    mesh = plsc.VectorSubcoreMesh(core_axis_name="c", subcore_axis_name="s")
    f = pl.kernel(
        _sc_body,
        out_type=jax.ShapeDtypeStruct((77, _N), jnp.float32),
        mesh=mesh,
        scratch_types=[
            pltpu.VMEM((2, 3, _E), jnp.int32),
            pltpu.VMEM((2, 77, _E), jnp.float32),
            pltpu.VMEM((256, 64), jnp.float32),
            pltpu.VMEM((256, _FWIDTH), jnp.float32),
            pltpu.SemaphoreType.DMA,
            pltpu.SemaphoreType.DMA,
            pltpu.SemaphoreType.DMA,
            pltpu.SemaphoreType.DMA,
        ],
        compiler_params=pltpu.CompilerParams(
            use_tc_tiling_on_sc=False, needs_layout_passes=False),
    )
    return f(td_lin, w, txy)


def kernel(td, W):
    td_lin = (td.transpose(2, 1, 0)
                .reshape(3, _S // 8, 8, _B // 128, 128)
                .transpose(0, 1, 3, 2, 4)
                .reshape(3, _N))
    k5 = _run(td_lin, W, jnp.asarray(_TXY))
    out = (k5.reshape(77, _S // 8, _B // 128, 8, 128)
             .transpose(2, 4, 1, 3, 0)
             .reshape(_B, _S, 77))
    return out
